# baseline (speedup 1.0000x reference)
import functools

import jax
import jax.numpy as jnp
from jax import lax
from jax.experimental import pallas as pl
from jax.experimental.pallas import tpu as pltpu
from jax.experimental.pallas import tpu_sc as plsc

NUM_ITEMS = 1000000
EMB = 64
BATCH = 4096
HIST = 200
NW = 32
ROWS_W = BATCH // NW
NB = 1
NCH = ROWS_W // NB


def _emb_body(idx_hbm, tab_hbm, out_hbm, idx_v, rows_v, sg0, sg1, ss0, ss1):
    wid = lax.axis_index("s") * 2 + lax.axis_index("c")
    base = wid * ROWS_W

    pltpu.sync_copy(idx_hbm.at[pl.ds(base, ROWS_W)], idx_v)

    sg = (sg0, sg1)
    ss = (ss0, ss1)

    def start_gather(i, b):
        pltpu.async_copy(
            tab_hbm.at[idx_v.at[i]], rows_v.at[b], sg[b])

    def wait_gather(i, b):
        pltpu.make_async_copy(
            tab_hbm.at[idx_v.at[i]], rows_v.at[b],
            sg[b]).wait()

    def start_store(i, b):
        pltpu.async_copy(
            rows_v.at[b], out_hbm.at[base + i], ss[b])

    def wait_store(i, b):
        pltpu.make_async_copy(
            rows_v.at[b], out_hbm.at[base + i],
            ss[b]).wait()

    start_gather(0, 0)
    wait_gather(0, 0)
    start_gather(1, 1)
    start_store(0, 0)

    @pl.loop(0, (NCH - 2) // 2)
    def _(j):
        i = 1 + 2 * j
        wait_gather(i, 1)
        wait_store(i - 1, 0)
        start_gather(i + 1, 0)
        start_store(i, 1)
        wait_gather(i + 1, 0)
        wait_store(i, 1)
        start_gather(i + 2, 1)
        start_store(i + 1, 0)

    wait_gather(NCH - 1, 1)
    wait_store(NCH - 2, 0)
    start_store(NCH - 1, 1)
    wait_store(NCH - 1, 1)


@jax.jit
def _emb_lookup(idx2d, item_emb):
    mesh = plsc.VectorSubcoreMesh(core_axis_name="c", subcore_axis_name="s")
    f = functools.partial(
        pl.kernel,
        out_type=jax.ShapeDtypeStruct((BATCH, HIST, EMB), jnp.float32),
        mesh=mesh,
        compiler_params=pltpu.CompilerParams(use_tc_tiling_on_sc=False),
        scratch_types=[
            pltpu.VMEM((ROWS_W, HIST), jnp.int32),
            pltpu.VMEM((2, HIST, EMB), jnp.float32),
            pltpu.SemaphoreType.DMA,
            pltpu.SemaphoreType.DMA,
            pltpu.SemaphoreType.DMA,
            pltpu.SemaphoreType.DMA,
        ],
    )(_emb_body)
    return f(idx2d, item_emb)


def kernel(input_seqs, item_emb):
    return _emb_lookup(input_seqs, item_emb)


# speedup vs baseline: 1.2447x; 1.2447x over previous
import functools

import jax
import jax.numpy as jnp
from jax import lax
from jax.experimental import pallas as pl
from jax.experimental.pallas import tpu as pltpu
from jax.experimental.pallas import tpu_sc as plsc

NUM_ITEMS = 1000000
EMB = 64
BATCH = 4096
HIST = 200
NW = 32
ROWS_W = BATCH // NW
PER_W = ROWS_W * HIST


def _emb_body(idx_hbm, tab_hbm, out_hbm, idx_v, rows_v, sg0, sg1, ss0, ss1):
    wid = lax.axis_index("s") * 2 + lax.axis_index("c")
    base = wid * ROWS_W

    pltpu.sync_copy(idx_hbm.at[wid], idx_v)

    sg = (sg0, sg1)
    ss = (ss0, ss1)

    def start_gather(i, b):
        pltpu.async_copy(
            tab_hbm.at[idx_v.at[pl.ds(i * HIST, HIST)]], rows_v.at[b], sg[b])

    def wait_gather(i, b):
        pltpu.make_async_copy(
            tab_hbm.at[idx_v.at[pl.ds(i * HIST, HIST)]], rows_v.at[b],
            sg[b]).wait()

    def start_store(i, b):
        pltpu.async_copy(rows_v.at[b], out_hbm.at[base + i], ss[b])

    def wait_store(i, b):
        pltpu.make_async_copy(rows_v.at[b], out_hbm.at[base + i],
                              ss[b]).wait()

    start_gather(0, 0)
    wait_gather(0, 0)
    start_gather(1, 1)
    start_store(0, 0)

    @pl.loop(0, (ROWS_W - 2) // 2)
    def _(j):
        i = 1 + 2 * j
        wait_gather(i, 1)
        wait_store(i - 1, 0)
        start_gather(i + 1, 0)
        start_store(i, 1)
        wait_gather(i + 1, 0)
        wait_store(i, 1)
        start_gather(i + 2, 1)
        start_store(i + 1, 0)

    wait_gather(ROWS_W - 1, 1)
    wait_store(ROWS_W - 2, 0)
    start_store(ROWS_W - 1, 1)
    wait_store(ROWS_W - 1, 1)


@jax.jit
def _emb_lookup(idx32, tab128):
    mesh = plsc.VectorSubcoreMesh(core_axis_name="c", subcore_axis_name="s")
    f = functools.partial(
        pl.kernel,
        out_type=jax.ShapeDtypeStruct((BATCH, HIST, 128), jnp.float32),
        mesh=mesh,
        compiler_params=pltpu.CompilerParams(use_tc_tiling_on_sc=True),
        scratch_types=[
            pltpu.VMEM((PER_W,), jnp.int32),
            pltpu.VMEM((2, HIST, 128), jnp.float32),
            pltpu.SemaphoreType.DMA,
            pltpu.SemaphoreType.DMA,
            pltpu.SemaphoreType.DMA,
            pltpu.SemaphoreType.DMA,
        ],
    )(_emb_body)
    return f(idx32, tab128)


def kernel(input_seqs, item_emb):
    tab128 = jnp.pad(item_emb, ((0, 0), (0, 128 - EMB)))
    idx32 = input_seqs.reshape(NW, PER_W)
    return _emb_lookup(idx32, tab128)[..., :EMB]
